# baseline (speedup 1.0000x reference)
import functools

import jax
import jax.numpy as jnp
from jax import lax
from jax.experimental import pallas as pl
from jax.experimental.pallas import tpu as pltpu
from jax.experimental.pallas import tpu_sc as plsc

VOCAB = 100000
EMBED = 32
BATCH = 1024


_info = plsc.get_sparse_core_info()
_NC, _NS, _NL = _info.num_cores, _info.num_subcores, _info.num_lanes
_NW = _NC * _NS


def _make_sc_gather_t():
    mesh = plsc.VectorSubcoreMesh(core_axis_name="c", subcore_axis_name="s")

    @functools.partial(
        pl.kernel,
        mesh=mesh,
        compiler_params=pltpu.CompilerParams(needs_layout_passes=False),
        out_type=jax.ShapeDtypeStruct((EMBED, BATCH), jnp.float32),
        scratch_types=[
            pltpu.VMEM((VOCAB,), jnp.float32),
            pltpu.VMEM((BATCH,), jnp.int32),
            pltpu.VMEM((BATCH,), jnp.float32),
            pltpu.SemaphoreType.DMA,
        ],
    )
    def gather_kernel(et_hbm, idx_hbm, out_hbm, row_v, idx_v, hrow_v, sem):
        wid = lax.axis_index("s") * _NC + lax.axis_index("c")
        row_cp = pltpu.async_copy(et_hbm.at[wid], row_v, sem)
        pltpu.sync_copy(idx_hbm, idx_v)
        row_cp.wait()
        for g in range(BATCH // _NL):
            sl = pl.ds(g * _NL, _NL)
            hrow_v[sl] = plsc.load_gather(row_v, [idx_v[sl]])
        pltpu.sync_copy(hrow_v, out_hbm.at[wid])

    return gather_kernel


_sc_gather_t = _make_sc_gather_t()


_NV = 4096


def _proj_kernel(ht_ref, wt_ref, b_ref, out_ref):
    ht = ht_ref[...]
    wt = wt_ref[...]
    acc = lax.dot_general(
        wt, ht,
        dimension_numbers=(((0,), (0,)), ((), ())),
        preferred_element_type=jnp.float32,
    )
    bias = b_ref[...].reshape(_NV, 1)
    out_ref[...] = acc + bias


def _projection_t(ht, Wt, b2d):
    grid = (pl.cdiv(VOCAB, _NV),)
    return pl.pallas_call(
        _proj_kernel,
        grid=grid,
        in_specs=[
            pl.BlockSpec((EMBED, BATCH), lambda j: (0, 0)),
            pl.BlockSpec((EMBED, _NV), lambda j: (0, j)),
            pl.BlockSpec((1, _NV), lambda j: (0, j)),
        ],
        out_specs=pl.BlockSpec((_NV, BATCH), lambda j: (j, 0)),
        out_shape=jax.ShapeDtypeStruct((VOCAB, BATCH), jnp.float32),
        compiler_params=pltpu.CompilerParams(
            dimension_semantics=("arbitrary",),
        ),
    )(ht, Wt, b2d)


def kernel(x, emb_table, W, b):
    ht = _sc_gather_t(emb_table.T, x)
    out_t = _projection_t(ht, W.T, b.reshape(1, VOCAB))
    return out_t.T


# speedup vs baseline: 1.0032x; 1.0032x over previous
import functools

import jax
import jax.numpy as jnp
from jax import lax
from jax.experimental import pallas as pl
from jax.experimental.pallas import tpu as pltpu
from jax.experimental.pallas import tpu_sc as plsc

VOCAB = 100000
EMBED = 32
BATCH = 1024


_info = plsc.get_sparse_core_info()
_NC, _NS, _NL = _info.num_cores, _info.num_subcores, _info.num_lanes
_NW = _NC * _NS
assert _NW == EMBED and BATCH % _NL == 0


def _make_sc_gather_t():
    mesh = plsc.VectorSubcoreMesh(core_axis_name="c", subcore_axis_name="s")

    @functools.partial(
        pl.kernel,
        mesh=mesh,
        compiler_params=pltpu.CompilerParams(needs_layout_passes=False),
        out_type=jax.ShapeDtypeStruct((EMBED, BATCH), jnp.float32),
        scratch_types=[
            pltpu.VMEM((VOCAB,), jnp.float32),
            pltpu.VMEM((BATCH,), jnp.int32),
            pltpu.VMEM((BATCH,), jnp.float32),
            pltpu.SemaphoreType.DMA,
        ],
    )
    def gather_kernel(et_hbm, idx_hbm, out_hbm, row_v, idx_v, hrow_v, sem):
        wid = lax.axis_index("s") * _NC + lax.axis_index("c")
        row_cp = pltpu.async_copy(et_hbm.at[wid], row_v, sem)
        pltpu.sync_copy(idx_hbm, idx_v)
        row_cp.wait()
        for g in range(BATCH // _NL):
            sl = pl.ds(g * _NL, _NL)
            hrow_v[sl] = plsc.load_gather(row_v, [idx_v[sl]])
        pltpu.sync_copy(hrow_v, out_hbm.at[wid])

    return gather_kernel


_sc_gather_t = _make_sc_gather_t()


_NV = 4096


def _proj_kernel(ht_ref, wt_ref, b_ref, out_ref):
    ht = ht_ref[...]
    wt = wt_ref[...]
    acc = lax.dot_general(
        wt, ht,
        dimension_numbers=(((0,), (0,)), ((), ())),
        preferred_element_type=jnp.float32,
    )
    bias = b_ref[...].reshape(_NV, 1)
    out_ref[...] = acc + bias


def _projection_t(ht, Wt, b2d):
    grid = (pl.cdiv(VOCAB, _NV),)
    return pl.pallas_call(
        _proj_kernel,
        grid=grid,
        in_specs=[
            pl.BlockSpec((EMBED, BATCH), lambda j: (0, 0)),
            pl.BlockSpec((EMBED, _NV), lambda j: (0, j)),
            pl.BlockSpec((1, _NV), lambda j: (0, j)),
        ],
        out_specs=pl.BlockSpec((_NV, BATCH), lambda j: (j, 0)),
        out_shape=jax.ShapeDtypeStruct((VOCAB, BATCH), jnp.float32),
        compiler_params=pltpu.CompilerParams(
            dimension_semantics=("arbitrary",),
        ),
    )(ht, Wt, b2d)


def kernel(x, emb_table, W, b):
    ht = _sc_gather_t(emb_table.T, x)
    out_t = _projection_t(ht, W.T, b.reshape(1, VOCAB))
    return out_t.T
